# Initial kernel scaffold; baseline (speedup 1.0000x reference)
import jax
import jax.numpy as jnp
from jax.experimental import pallas as pl


def kernel(x, edge_index, W1, b1, g1, bt1, rm1, rv1, W2, b2, g2, bt2, rm2, rv2):
    raise NotImplementedError("write your pallas kernel here")


# speedup vs baseline: 7.5098x; 7.5098x over previous
import functools

import jax
import jax.numpy as jnp
from jax import lax
from jax.experimental import pallas as pl
from jax.experimental.pallas import tpu as pltpu
from jax.experimental.pallas import tpu_sc as plsc

N = 10000
N_P = 10112
E = 320000
D = 128
EPS = 1e-5

NC = 2
NS = 16
NW = NC * NS
CH = 128
CPW = 80
E_PAD = NW * CPW * CH
ROWS_PER_TILE = N_P // NS
DEG_PAD = 10240
DEG_PER_TILE = DEG_PAD // NS

_mesh = plsc.VectorSubcoreMesh(core_axis_name="c", subcore_axis_name="s")


@functools.partial(
    pl.kernel,
    out_type=jax.ShapeDtypeStruct((NC, DEG_PAD), jnp.float32),
    mesh=_mesh,
    scratch_types=[
        pltpu.VMEM_SHARED((DEG_PAD,), jnp.float32),
        pltpu.VMEM((CPW, CH), jnp.int32),
        pltpu.VMEM((CH,), jnp.float32),
        pltpu.VMEM((DEG_PER_TILE,), jnp.float32),
        pltpu.SemaphoreType.DMA,
    ],
)
def _deg_kernel(row2_hbm, out_hbm, deg_acc, ridx, ones_v, zer_v, sem):
    c = lax.axis_index("c")
    s = lax.axis_index("s")
    wid = s * NC + c
    for i in range(CH // 16):
        ones_v[pl.ds(i * 16, 16)] = jnp.full((16,), 1.0, jnp.float32)
    for i in range(DEG_PER_TILE // 16):
        zer_v[pl.ds(i * 16, 16)] = jnp.zeros((16,), jnp.float32)
    pltpu.sync_copy(zer_v, deg_acc.at[pl.ds(s * DEG_PER_TILE, DEG_PER_TILE)])
    pltpu.sync_copy(row2_hbm.at[pl.ds(wid * CPW, CPW)], ridx)
    plsc.subcore_barrier()

    def fire(j, carry):
        pltpu.async_copy(ones_v, deg_acc.at[ridx.at[j]], sem, add=True)
        return carry

    lax.fori_loop(0, CPW, fire, 0)

    def drain(j, carry):
        pltpu.make_async_copy(ones_v, deg_acc.at[ridx.at[0]], sem).wait()
        return carry

    lax.fori_loop(0, CPW, drain, 0)
    plsc.subcore_barrier()
    pltpu.sync_copy(deg_acc.at[pl.ds(s * DEG_PER_TILE, DEG_PER_TILE)],
                    out_hbm.at[c, pl.ds(s * DEG_PER_TILE, DEG_PER_TILE)])


@functools.partial(
    pl.kernel,
    out_type=jax.ShapeDtypeStruct((NC, N_P, D), jnp.float32),
    mesh=_mesh,
    scratch_types=[
        pltpu.VMEM_SHARED((N_P, D), jnp.float32),
        pltpu.VMEM((CPW, CH), jnp.int32),
        pltpu.VMEM((8, CH), jnp.int32),
        pltpu.VMEM((CH, D), jnp.float32),
        pltpu.VMEM((CH, D), jnp.float32),
        pltpu.SemaphoreType.DMA,
        pltpu.SemaphoreType.DMA,
    ],
)
def _mp_kernel(table_hbm, row2_hbm, col2_hbm, out_hbm,
               s_acc, cidx, ridx_blk, msgs_a, msgs_b, sem_a, sem_b):
    c = lax.axis_index("c")
    s = lax.axis_index("s")
    wid = s * NC + c
    pltpu.sync_copy(table_hbm.at[pl.ds(s * ROWS_PER_TILE, ROWS_PER_TILE)],
                    s_acc.at[pl.ds(s * ROWS_PER_TILE, ROWS_PER_TILE)])
    pltpu.sync_copy(col2_hbm.at[pl.ds(wid * CPW, CPW)], cidx)
    plsc.subcore_barrier()

    def chunk_group(g, carry):
        pltpu.sync_copy(row2_hbm.at[pl.ds(wid * CPW + 8 * g, 8)], ridx_blk)
        for k in range(0, 8, 2):
            a = pltpu.async_copy(table_hbm.at[cidx.at[8 * g + k]],
                                 msgs_a, sem_a)
            b = pltpu.async_copy(table_hbm.at[cidx.at[8 * g + k + 1]],
                                 msgs_b, sem_b)
            a.wait()
            pltpu.sync_copy(msgs_a, s_acc.at[ridx_blk.at[k]], add=True)
            b.wait()
            pltpu.sync_copy(msgs_b, s_acc.at[ridx_blk.at[k + 1]], add=True)
        return carry

    lax.fori_loop(0, CPW // 8, chunk_group, 0)
    plsc.subcore_barrier()
    pltpu.sync_copy(s_acc.at[pl.ds(s * ROWS_PER_TILE, ROWS_PER_TILE)],
                    out_hbm.at[c, pl.ds(s * ROWS_PER_TILE, ROWS_PER_TILE)])


_RB = 1264
_GRID = N_P // _RB


def _t1_body(dp_ref, x_ref, w_ref, tab_ref, dis_ref):
    deg = dp_ref[0, :, :] + dp_ref[1, :, :] + 1.0
    dis = lax.rsqrt(deg)
    t = jnp.dot(x_ref[...], w_ref[...], preferred_element_type=jnp.float32)
    tab_ref[...] = dis * t
    dis_ref[...] = dis


def _table1(dp3, x, W1):
    return pl.pallas_call(
        _t1_body,
        grid=(_GRID,),
        in_specs=[
            pl.BlockSpec((2, _RB, 1), lambda i: (0, i, 0)),
            pl.BlockSpec((_RB, D), lambda i: (i, 0)),
            pl.BlockSpec((D, D), lambda i: (0, 0)),
        ],
        out_specs=[
            pl.BlockSpec((_RB, D), lambda i: (i, 0)),
            pl.BlockSpec((_RB, 1), lambda i: (i, 0)),
        ],
        out_shape=[
            jax.ShapeDtypeStruct((N_P, D), jnp.float32),
            jax.ShapeDtypeStruct((N_P, 1), jnp.float32),
        ],
    )(dp3, x, W1)


def _bn_elu(h, g, bt, rm, rv):
    xn = (h - rm) * (g * lax.rsqrt(rv + EPS)) + bt
    return jnp.where(xn > 0, xn, jnp.exp(xn) - 1.0)


def _mid_body(sp_ref, tab_ref, dis_ref, b_ref, g_ref, bt_ref, rm_ref, rv_ref,
              w_ref, out_ref):
    dis = dis_ref[...]
    comb = dis * (sp_ref[0, :, :] + sp_ref[1, :, :] - tab_ref[...]) + b_ref[...]
    h = _bn_elu(comb, g_ref[...], bt_ref[...], rm_ref[...], rv_ref[...])
    out_ref[...] = dis * jnp.dot(h, w_ref[...],
                                 preferred_element_type=jnp.float32)


def _table2(sp, tab1, dis, b, g, bt, rm, rv, W2):
    return pl.pallas_call(
        _mid_body,
        grid=(_GRID,),
        in_specs=[
            pl.BlockSpec((2, _RB, D), lambda i: (0, i, 0)),
            pl.BlockSpec((_RB, D), lambda i: (i, 0)),
            pl.BlockSpec((_RB, 1), lambda i: (i, 0)),
        ] + [pl.BlockSpec((1, D), lambda i: (0, 0))] * 5 + [
            pl.BlockSpec((D, D), lambda i: (0, 0)),
        ],
        out_specs=pl.BlockSpec((_RB, D), lambda i: (i, 0)),
        out_shape=jax.ShapeDtypeStruct((N_P, D), jnp.float32),
    )(sp, tab1, dis, b, g, bt, rm, rv, W2)


def _out_body(sp_ref, tab_ref, dis_ref, b_ref, g_ref, bt_ref, rm_ref, rv_ref,
              out_ref):
    comb = (dis_ref[...] * (sp_ref[0, :, :] + sp_ref[1, :, :] - tab_ref[...])
            + b_ref[...])
    out_ref[...] = _bn_elu(comb, g_ref[...], bt_ref[...], rm_ref[...],
                           rv_ref[...])


def _final(sp, tab2, dis, b, g, bt, rm, rv):
    return pl.pallas_call(
        _out_body,
        grid=(_GRID,),
        in_specs=[
            pl.BlockSpec((2, _RB, D), lambda i: (0, i, 0)),
            pl.BlockSpec((_RB, D), lambda i: (i, 0)),
            pl.BlockSpec((_RB, 1), lambda i: (i, 0)),
        ] + [pl.BlockSpec((1, D), lambda i: (0, 0))] * 5,
        out_specs=pl.BlockSpec((_RB, D), lambda i: (i, 0)),
        out_shape=jax.ShapeDtypeStruct((N_P, D), jnp.float32),
    )(sp, tab2, dis, b, g, bt, rm, rv)


def kernel(x, edge_index, W1, b1, g1, bt1, rm1, rv1, W2, b2, g2, bt2, rm2, rv2):
    row = edge_index[0]
    col = edge_index[1]
    pad = E_PAD - E
    rowp = jnp.concatenate(
        [row, jnp.full((pad,), N, jnp.int32)]).reshape(E_PAD // CH, CH)
    colp = jnp.concatenate(
        [col, jnp.zeros((pad,), jnp.int32)]).reshape(E_PAD // CH, CH)
    xp = jnp.concatenate(
        [x, jnp.zeros((N_P - N, D), jnp.float32)], axis=0)

    degp = _deg_kernel(rowp)
    dp3 = degp.reshape(NC, DEG_PAD, 1)

    b1r, g1r, bt1r, rm1r, rv1r = (v.reshape(1, D) for v in (b1, g1, bt1, rm1, rv1))
    b2r, g2r, bt2r, rm2r, rv2r = (v.reshape(1, D) for v in (b2, g2, bt2, rm2, rv2))

    tab1, dis = _table1(dp3, xp, W1)
    sp1 = _mp_kernel(tab1, rowp, colp)
    tab2 = _table2(sp1, tab1, dis, b1r, g1r, bt1r, rm1r, rv1r, W2)
    sp2 = _mp_kernel(tab2, rowp, colp)
    out = _final(sp2, tab2, dis, b2r, g2r, bt2r, rm2r, rv2r)
    return out[:N]
